# baseline (speedup 1.0000x reference)
import functools

import jax
import jax.numpy as jnp
from jax import lax
from jax.experimental import pallas as pl
from jax.experimental.pallas import tpu as pltpu
from jax.experimental.pallas import tpu_sc as plsc

N = 10000
E = 320000
NP = 10240
SL = NP // 16
ET = E // 16
EWK = 2 * E // 32
CH = 80
NCH = EWK // CH
LANES = 16


def _node_mm_body(x_ref, w_ref, o_ref):
    o_ref[...] = jnp.dot(x_ref[...], w_ref[...],
                         preferred_element_type=jnp.float32)


def _node_matmul(x, w):
    blk = 1000
    return pl.pallas_call(
        _node_mm_body,
        grid=(N // blk,),
        in_specs=[pl.BlockSpec((blk, 128), lambda i: (i, 0)),
                  pl.BlockSpec((128, 32), lambda i: (0, 0))],
        out_specs=pl.BlockSpec((blk, 32), lambda i: (i, 0)),
        out_shape=jax.ShapeDtypeStruct((N, 32), jnp.float32),
    )(x, w)


def _gather_sum_body(xa_hbm, xb_hbm, src_hbm, dst_hbm, out_hbm,
                     sidx, didx, abuf, bbuf, obuf, sema, semb):
    c = lax.axis_index("c")
    s = lax.axis_index("s")
    wid = c * 16 + s
    base = wid * EWK
    pltpu.sync_copy(src_hbm.at[pl.ds(base, EWK)], sidx)
    pltpu.sync_copy(dst_hbm.at[pl.ds(base, EWK)], didx)

    def fire(g, slot):
        pltpu.async_copy(xa_hbm.at[sidx.at[pl.ds(g * CH, CH)]],
                         abuf.at[slot], sema)
        pltpu.async_copy(xb_hbm.at[didx.at[pl.ds(g * CH, CH)]],
                         bbuf.at[slot], semb)

    fire(0, 0)
    fire(1, 1)

    rows16 = lax.iota(jnp.int32, LANES)

    def outer(i, carry):
        for b in range(2):
            g = i * 2 + b
            pltpu.make_async_copy(xa_hbm.at[sidx.at[pl.ds(0, CH)]],
                                  abuf.at[b], sema).wait()
            pltpu.make_async_copy(xb_hbm.at[didx.at[pl.ds(0, CH)]],
                                  bbuf.at[b], semb).wait()
            for r in range(CH):
                plsc.store_scatter(
                    obuf.at[b], [rows16, jnp.full((LANES,), r, jnp.int32)],
                    abuf[b, r] + bbuf[b, r])
            pltpu.sync_copy(obuf.at[b],
                            out_hbm.at[:, pl.ds(base + g * CH, CH)])

            @pl.when(g + 2 < NCH)
            def _():
                fire(g + 2, b)
        return carry

    lax.fori_loop(0, NCH // 2, outer, 0)


def _gather_sum(xa, xb, src, dst):
    mesh = plsc.VectorSubcoreMesh(core_axis_name="c", subcore_axis_name="s")
    f = pl.kernel(
        _gather_sum_body,
        out_type=jax.ShapeDtypeStruct((16, 2 * E), jnp.float32),
        mesh=mesh,
        compiler_params=pltpu.CompilerParams(use_tc_tiling_on_sc=False, needs_layout_passes=False),
        scratch_types=[
            pltpu.VMEM((EWK,), jnp.int32),
            pltpu.VMEM((EWK,), jnp.int32),
            pltpu.VMEM((2, CH, 16), jnp.float32),
            pltpu.VMEM((2, CH, 16), jnp.float32),
            pltpu.VMEM((2, 16, CH), jnp.float32),
            pltpu.SemaphoreType.DMA,
            pltpu.SemaphoreType.DMA,
        ],
    )
    return f(xa, xb, src, dst)


def _mlp_body(t_ref, a_ref, bt_ref, w1aT_ref, w1bT_ref, b1_ref, w2T_ref,
              b2_ref, o_ref):
    trans = jax.nn.relu(t_ref[...] + bt_ref[...])
    h = jax.nn.relu(
        jnp.dot(w1aT_ref[...], a_ref[...],
                preferred_element_type=jnp.float32)
        + jnp.dot(w1bT_ref[...], trans, preferred_element_type=jnp.float32)
        + b1_ref[...])
    o_ref[...] = jax.nn.sigmoid(
        jnp.dot(w2T_ref[...], h, preferred_element_type=jnp.float32)
        + b2_ref[...])


def _edge_mlp(tsumT, aT, btc, w1aT, w1bT, b1c, w2T, b2c, col0):
    blkc = 2560
    nblk = E // blkc
    aw = aT.shape[0]
    return pl.pallas_call(
        _mlp_body,
        grid=(nblk,),
        in_specs=[
            pl.BlockSpec((16, blkc), lambda i: (0, col0 + i)),
            pl.BlockSpec((aw, blkc), lambda i: (0, i)),
            pl.BlockSpec((16, 1), lambda i: (0, 0)),
            pl.BlockSpec((32, aw), lambda i: (0, 0)),
            pl.BlockSpec((32, 16), lambda i: (0, 0)),
            pl.BlockSpec((32, 1), lambda i: (0, 0)),
            pl.BlockSpec((1, 32), lambda i: (0, 0)),
            pl.BlockSpec((1, 1), lambda i: (0, 0)),
        ],
        out_specs=pl.BlockSpec((1, blkc), lambda i: (0, i)),
        out_shape=jax.ShapeDtypeStruct((1, E), jnp.float32),
    )(tsumT, aT, btc, w1aT, w1bT, b1c, w2T, b2c)


def _prop_body(src_hbm, dst_hbm, ew_hbm, m0_hbm, out_hbm,
               isrc, idst, wv, m_in, m_out, mrg, msl, sh_all, sh_merged):
    c = lax.axis_index("c")
    s = lax.axis_index("s")
    base = s * ET
    pltpu.sync_copy(src_hbm.at[c, pl.ds(base, ET)], isrc)
    pltpu.sync_copy(dst_hbm.at[c, pl.ds(base, ET)], idst)
    pltpu.sync_copy(ew_hbm.at[c, pl.ds(base, ET)], wv)
    pltpu.sync_copy(m0_hbm, m_in)

    def copy_m(i, carry):
        k = i * LANES
        m_out[pl.ds(k, LANES)] = m_in[pl.ds(k, LANES)]
        return carry

    def edge(i, carry):
        k = i * LANES
        si = isrc[pl.ds(k, LANES)]
        di = idst[pl.ds(k, LANES)]
        v = wv[pl.ds(k, LANES)] * plsc.load_gather(m_in, [si])

        def wbody(act):
            cur = plsc.load_gather(m_out, [di])
            plsc.store_scatter(m_out, [di], jnp.maximum(cur, v), mask=act)
            chk = plsc.load_gather(m_out, [di])
            return jnp.logical_and(act, chk < v)

        lax.while_loop(lambda a: jnp.any(a), wbody,
                       jnp.ones((LANES,), jnp.bool_))
        return carry

    def reduce_slice(i, carry):
        k = i * LANES
        acc = mrg[0, pl.ds(k, LANES)]
        for t in range(1, 16):
            acc = jnp.maximum(acc, mrg[t, pl.ds(k, LANES)])
        msl[pl.ds(k, LANES)] = acc
        return carry

    for rnd in range(3):
        lax.fori_loop(0, NP // LANES, copy_m, 0)
        lax.fori_loop(0, ET // LANES, edge, 0)
        pltpu.sync_copy(m_out, sh_all.at[s])
        plsc.subcore_barrier()
        for t in range(16):
            pltpu.sync_copy(sh_all.at[t, pl.ds(s * SL, SL)], mrg.at[t])
        lax.fori_loop(0, SL // LANES, reduce_slice, 0)
        if rnd < 2:
            pltpu.sync_copy(msl, sh_merged.at[pl.ds(s * SL, SL)])
            plsc.subcore_barrier()
            pltpu.sync_copy(sh_merged, m_in)
        else:
            pltpu.sync_copy(msl, out_hbm.at[c, pl.ds(s * SL, SL)])


def _propagate(src2, dst2, ew2, m0p):
    mesh = plsc.VectorSubcoreMesh(core_axis_name="c", subcore_axis_name="s")
    f = pl.kernel(
        _prop_body,
        out_type=jax.ShapeDtypeStruct((2, NP), jnp.float32),
        mesh=mesh,
        compiler_params=pltpu.CompilerParams(use_tc_tiling_on_sc=False, needs_layout_passes=False),
        scratch_types=[
            pltpu.VMEM((ET,), jnp.int32),
            pltpu.VMEM((ET,), jnp.int32),
            pltpu.VMEM((ET,), jnp.float32),
            pltpu.VMEM((NP,), jnp.float32),
            pltpu.VMEM((NP,), jnp.float32),
            pltpu.VMEM((16, SL), jnp.float32),
            pltpu.VMEM((SL,), jnp.float32),
            pltpu.VMEM_SHARED((16, NP), jnp.float32),
            pltpu.VMEM_SHARED((NP,), jnp.float32),
        ],
    )
    return f(src2, dst2, ew2, m0p)


def kernel(x, spatial_edge_index, spatial_edge_attr, dom_edge_index,
           dom_edge_attr, mask, Wt, bt, Wp1, bp1, Wp2, bp2, Wd1, bd1,
           Wd2, bd2):
    w = jnp.concatenate([Wt[:128], Wt[128:]], axis=1)
    xw = _node_matmul(x, w)
    xa = jnp.pad(xw[:, :16], ((0, NP - N), (0, 0)))
    xb = jnp.pad(xw[:, 16:], ((0, NP - N), (0, 0)))

    src = jnp.concatenate([spatial_edge_index[0], dom_edge_index[0]])
    dst = jnp.concatenate([spatial_edge_index[1], dom_edge_index[1]])

    tsum = _gather_sum(xa, xb, src, dst)

    btc = bt.reshape(16, 1)
    ew_s = _edge_mlp(tsum, spatial_edge_attr.T, btc, Wp1[:4].T, Wp1[4:].T,
                     bp1.reshape(32, 1), Wp2.T, bp2.reshape(1, 1), 0)
    ew_d = _edge_mlp(tsum, dom_edge_attr.T, btc, Wd1[:1].T, Wd1[1:].T,
                     bd1.reshape(32, 1), Wd2.T, bd2.reshape(1, 1), E // 2560)
    ew2 = jnp.concatenate([ew_s, ew_d], axis=0)

    src2 = src.reshape(2, E)
    dst2 = dst.reshape(2, E)
    m0p = jnp.pad(mask, (0, NP - N))
    mout = _propagate(src2, dst2, ew2, m0p)

    return jnp.maximum(mout[0, :N], mout[1, :N])


# speedup vs baseline: 1.7423x; 1.7423x over previous
import functools

import jax
import jax.numpy as jnp
from jax import lax
from jax.experimental import pallas as pl
from jax.experimental.pallas import tpu as pltpu
from jax.experimental.pallas import tpu_sc as plsc

N = 10000
E = 320000
NP = 10240
SL = NP // 16
ET = E // 16
EWK = 2 * E // 32
CH = 80
NCH = EWK // CH
LANES = 16


def _node_mm_body(x_ref, w_ref, o_ref):
    o_ref[...] = jnp.dot(x_ref[...], w_ref[...],
                         preferred_element_type=jnp.float32)


def _node_matmul(x, w):
    blk = 1000
    return pl.pallas_call(
        _node_mm_body,
        grid=(N // blk,),
        in_specs=[pl.BlockSpec((blk, 128), lambda i: (i, 0)),
                  pl.BlockSpec((128, 32), lambda i: (0, 0))],
        out_specs=pl.BlockSpec((blk, 32), lambda i: (i, 0)),
        out_shape=jax.ShapeDtypeStruct((N, 32), jnp.float32),
    )(x, w)


def _gather_sum_body(xa_hbm, xb_hbm, src_hbm, dst_hbm, out_hbm,
                     sidx, didx, abuf, bbuf, obuf, sema, semb):
    c = lax.axis_index("c")
    s = lax.axis_index("s")
    wid = c * 16 + s
    base = wid * EWK
    pltpu.sync_copy(src_hbm.at[pl.ds(base, EWK)], sidx)
    pltpu.sync_copy(dst_hbm.at[pl.ds(base, EWK)], didx)

    def fire(g, slot):
        pltpu.async_copy(xa_hbm.at[sidx.at[pl.ds(g * CH, CH)]],
                         abuf.at[slot], sema)
        pltpu.async_copy(xb_hbm.at[didx.at[pl.ds(g * CH, CH)]],
                         bbuf.at[slot], semb)

    fire(0, 0)
    fire(1, 1)

    def outer(i, carry):
        for b in range(2):
            g = i * 2 + b
            pltpu.make_async_copy(xa_hbm.at[sidx.at[pl.ds(0, CH)]],
                                  abuf.at[b], sema).wait()
            pltpu.make_async_copy(xb_hbm.at[didx.at[pl.ds(0, CH)]],
                                  bbuf.at[b], semb).wait()
            for r in range(CH):
                obuf[b, r] = abuf[b, r] + bbuf[b, r]
            pltpu.sync_copy(obuf.at[b],
                            out_hbm.at[pl.ds(base + g * CH, CH)])

            @pl.when(g + 2 < NCH)
            def _():
                fire(g + 2, b)
        return carry

    lax.fori_loop(0, NCH // 2, outer, 0)


def _gather_sum(xa, xb, src, dst):
    mesh = plsc.VectorSubcoreMesh(core_axis_name="c", subcore_axis_name="s")
    f = pl.kernel(
        _gather_sum_body,
        out_type=jax.ShapeDtypeStruct((2 * E, 16), jnp.float32),
        mesh=mesh,
        compiler_params=pltpu.CompilerParams(use_tc_tiling_on_sc=False, needs_layout_passes=False),
        scratch_types=[
            pltpu.VMEM((EWK,), jnp.int32),
            pltpu.VMEM((EWK,), jnp.int32),
            pltpu.VMEM((2, CH, 16), jnp.float32),
            pltpu.VMEM((2, CH, 16), jnp.float32),
            pltpu.VMEM((2, CH, 16), jnp.float32),
            pltpu.SemaphoreType.DMA,
            pltpu.SemaphoreType.DMA,
        ],
    )
    return f(xa, xb, src, dst)


def _mlp_body(t_ref, a_ref, bt_ref, w1aT_ref, w1bT_ref, b1_ref, w2T_ref,
              b2_ref, o_ref):
    trans = jax.nn.relu(t_ref[...] + bt_ref[...])
    h = jax.nn.relu(
        jnp.dot(w1aT_ref[...], a_ref[...],
                preferred_element_type=jnp.float32)
        + jnp.dot(w1bT_ref[...], trans, preferred_element_type=jnp.float32)
        + b1_ref[...])
    o_ref[...] = jax.nn.sigmoid(
        jnp.dot(w2T_ref[...], h, preferred_element_type=jnp.float32)
        + b2_ref[...])


def _edge_mlp(tsumT, aT, btc, w1aT, w1bT, b1c, w2T, b2c, col0):
    blkc = 2560
    nblk = E // blkc
    aw = aT.shape[0]
    return pl.pallas_call(
        _mlp_body,
        grid=(nblk,),
        in_specs=[
            pl.BlockSpec((16, blkc), lambda i: (0, col0 + i)),
            pl.BlockSpec((aw, blkc), lambda i: (0, i)),
            pl.BlockSpec((16, 1), lambda i: (0, 0)),
            pl.BlockSpec((32, aw), lambda i: (0, 0)),
            pl.BlockSpec((32, 16), lambda i: (0, 0)),
            pl.BlockSpec((32, 1), lambda i: (0, 0)),
            pl.BlockSpec((1, 32), lambda i: (0, 0)),
            pl.BlockSpec((1, 1), lambda i: (0, 0)),
        ],
        out_specs=pl.BlockSpec((1, blkc), lambda i: (0, i)),
        out_shape=jax.ShapeDtypeStruct((1, E), jnp.float32),
    )(tsumT, aT, btc, w1aT, w1bT, b1c, w2T, b2c)


def _prop_body(src_hbm, dst_hbm, ew_hbm, m0_hbm, out_hbm,
               isrc, idst, wv, m_in, m_out, mrg, msl, sh_all, sh_merged):
    c = lax.axis_index("c")
    s = lax.axis_index("s")
    base = s * ET
    pltpu.sync_copy(src_hbm.at[c, pl.ds(base, ET)], isrc)
    pltpu.sync_copy(dst_hbm.at[c, pl.ds(base, ET)], idst)
    pltpu.sync_copy(ew_hbm.at[c, pl.ds(base, ET)], wv)
    pltpu.sync_copy(m0_hbm, m_in)

    def copy_m(i, carry):
        k = i * LANES
        m_out[pl.ds(k, LANES)] = m_in[pl.ds(k, LANES)]
        return carry

    def edge(i, carry):
        k = i * LANES
        si = isrc[pl.ds(k, LANES)]
        di = idst[pl.ds(k, LANES)]
        v = wv[pl.ds(k, LANES)] * plsc.load_gather(m_in, [si])

        def wbody(act):
            cur = plsc.load_gather(m_out, [di])
            plsc.store_scatter(m_out, [di], jnp.maximum(cur, v), mask=act)
            chk = plsc.load_gather(m_out, [di])
            return jnp.logical_and(act, chk < v)

        lax.while_loop(lambda a: jnp.any(a), wbody,
                       jnp.ones((LANES,), jnp.bool_))
        return carry

    def reduce_slice(i, carry):
        k = i * LANES
        acc = mrg[0, pl.ds(k, LANES)]
        for t in range(1, 16):
            acc = jnp.maximum(acc, mrg[t, pl.ds(k, LANES)])
        msl[pl.ds(k, LANES)] = acc
        return carry

    for rnd in range(3):
        lax.fori_loop(0, NP // LANES, copy_m, 0)
        lax.fori_loop(0, ET // LANES, edge, 0)
        pltpu.sync_copy(m_out, sh_all.at[s])
        plsc.subcore_barrier()
        for t in range(16):
            pltpu.sync_copy(sh_all.at[t, pl.ds(s * SL, SL)], mrg.at[t])
        lax.fori_loop(0, SL // LANES, reduce_slice, 0)
        if rnd < 2:
            pltpu.sync_copy(msl, sh_merged.at[pl.ds(s * SL, SL)])
            plsc.subcore_barrier()
            pltpu.sync_copy(sh_merged, m_in)
        else:
            pltpu.sync_copy(msl, out_hbm.at[c, pl.ds(s * SL, SL)])


def _propagate(src2, dst2, ew2, m0p):
    mesh = plsc.VectorSubcoreMesh(core_axis_name="c", subcore_axis_name="s")
    f = pl.kernel(
        _prop_body,
        out_type=jax.ShapeDtypeStruct((2, NP), jnp.float32),
        mesh=mesh,
        compiler_params=pltpu.CompilerParams(use_tc_tiling_on_sc=False, needs_layout_passes=False),
        scratch_types=[
            pltpu.VMEM((ET,), jnp.int32),
            pltpu.VMEM((ET,), jnp.int32),
            pltpu.VMEM((ET,), jnp.float32),
            pltpu.VMEM((NP,), jnp.float32),
            pltpu.VMEM((NP,), jnp.float32),
            pltpu.VMEM((16, SL), jnp.float32),
            pltpu.VMEM((SL,), jnp.float32),
            pltpu.VMEM_SHARED((16, NP), jnp.float32),
            pltpu.VMEM_SHARED((NP,), jnp.float32),
        ],
    )
    return f(src2, dst2, ew2, m0p)


def kernel(x, spatial_edge_index, spatial_edge_attr, dom_edge_index,
           dom_edge_attr, mask, Wt, bt, Wp1, bp1, Wp2, bp2, Wd1, bd1,
           Wd2, bd2):
    w = jnp.concatenate([Wt[:128], Wt[128:]], axis=1)
    xw = _node_matmul(x, w)
    xa = jnp.pad(xw[:, :16], ((0, NP - N), (0, 0)))
    xb = jnp.pad(xw[:, 16:], ((0, NP - N), (0, 0)))

    src = jnp.concatenate([spatial_edge_index[0], dom_edge_index[0]])
    dst = jnp.concatenate([spatial_edge_index[1], dom_edge_index[1]])

    tsum = _gather_sum(xa, xb, src, dst)

    tsumT = tsum.T
    btc = bt.reshape(16, 1)
    ew_s = _edge_mlp(tsumT, spatial_edge_attr.T, btc, Wp1[:4].T, Wp1[4:].T,
                     bp1.reshape(32, 1), Wp2.T, bp2.reshape(1, 1), 0)
    ew_d = _edge_mlp(tsumT, dom_edge_attr.T, btc, Wd1[:1].T, Wd1[1:].T,
                     bd1.reshape(32, 1), Wd2.T, bd2.reshape(1, 1), E // 2560)
    ew2 = jnp.concatenate([ew_s, ew_d], axis=0)

    src2 = src.reshape(2, E)
    dst2 = dst.reshape(2, E)
    m0p = jnp.pad(mask, (0, NP - N))
    mout = _propagate(src2, dst2, ew2, m0p)

    return jnp.maximum(mout[0, :N], mout[1, :N])


# speedup vs baseline: 1.7693x; 1.0155x over previous
import functools

import jax
import jax.numpy as jnp
from jax import lax
from jax.experimental import pallas as pl
from jax.experimental.pallas import tpu as pltpu
from jax.experimental.pallas import tpu_sc as plsc

N = 10000
E = 320000
NP = 10240
SL = NP // 16
ET = E // 16
EWK = 2 * E // 32
CH = 80
NCH = EWK // CH
LANES = 16


def _node_mm_body(x_ref, w_ref, o_ref):
    o_ref[...] = jnp.dot(x_ref[...], w_ref[...],
                         preferred_element_type=jnp.float32)


def _node_matmul(x, w):
    blk = 1000
    return pl.pallas_call(
        _node_mm_body,
        grid=(N // blk,),
        in_specs=[pl.BlockSpec((blk, 128), lambda i: (i, 0)),
                  pl.BlockSpec((128, 32), lambda i: (0, 0))],
        out_specs=pl.BlockSpec((blk, 32), lambda i: (i, 0)),
        out_shape=jax.ShapeDtypeStruct((N, 32), jnp.float32),
    )(x, w)


def _gather_sum_body(xa_hbm, xb_hbm, src_hbm, dst_hbm, out_hbm,
                     sidx, didx, abuf, bbuf, obuf, sema, semb, semo0, semo1):
    c = lax.axis_index("c")
    s = lax.axis_index("s")
    wid = c * 16 + s
    base = wid * EWK
    pltpu.sync_copy(src_hbm.at[pl.ds(base, EWK)], sidx)
    pltpu.sync_copy(dst_hbm.at[pl.ds(base, EWK)], didx)

    def fire(g, slot):
        pltpu.async_copy(xa_hbm.at[sidx.at[pl.ds(g * CH, CH)]],
                         abuf.at[slot], sema)
        pltpu.async_copy(xb_hbm.at[didx.at[pl.ds(g * CH, CH)]],
                         bbuf.at[slot], semb)

    fire(0, 0)
    fire(1, 1)

    def outer(i, carry):
        for b in range(2):
            g = i * 2 + b
            pltpu.make_async_copy(xa_hbm.at[sidx.at[pl.ds(0, CH)]],
                                  abuf.at[b], sema).wait()
            pltpu.make_async_copy(xb_hbm.at[didx.at[pl.ds(0, CH)]],
                                  bbuf.at[b], semb).wait()
            semo = semo0 if b == 0 else semo1

            @pl.when(g >= 2)
            def _():
                pltpu.make_async_copy(
                    obuf.at[b], out_hbm.at[pl.ds(base, CH)], semo).wait()

            for r in range(CH):
                obuf[b, r] = abuf[b, r] + bbuf[b, r]
            pltpu.async_copy(obuf.at[b],
                             out_hbm.at[pl.ds(base + g * CH, CH)], semo)

            @pl.when(g + 2 < NCH)
            def _():
                fire(g + 2, b)
        return carry

    lax.fori_loop(0, NCH // 2, outer, 0)
    for b in range(2):
        semo = semo0 if b == 0 else semo1
        pltpu.make_async_copy(
            obuf.at[b], out_hbm.at[pl.ds(base, CH)], semo).wait()


def _gather_sum(xa, xb, src, dst):
    mesh = plsc.VectorSubcoreMesh(core_axis_name="c", subcore_axis_name="s")
    f = pl.kernel(
        _gather_sum_body,
        out_type=jax.ShapeDtypeStruct((2 * E, 16), jnp.float32),
        mesh=mesh,
        compiler_params=pltpu.CompilerParams(use_tc_tiling_on_sc=False, needs_layout_passes=False),
        scratch_types=[
            pltpu.VMEM((EWK,), jnp.int32),
            pltpu.VMEM((EWK,), jnp.int32),
            pltpu.VMEM((2, CH, 16), jnp.float32),
            pltpu.VMEM((2, CH, 16), jnp.float32),
            pltpu.VMEM((2, CH, 16), jnp.float32),
            pltpu.SemaphoreType.DMA,
            pltpu.SemaphoreType.DMA,
            pltpu.SemaphoreType.DMA,
            pltpu.SemaphoreType.DMA,
        ],
    )
    return f(xa, xb, src, dst)


def _mlp_body(t_ref, a_ref, bt_ref, w1aT_ref, w1bT_ref, b1_ref, w2T_ref,
              b2_ref, o_ref):
    trans = jax.nn.relu(t_ref[...] + bt_ref[...])
    h = jax.nn.relu(
        jnp.dot(w1aT_ref[...], a_ref[...],
                preferred_element_type=jnp.float32)
        + jnp.dot(w1bT_ref[...], trans, preferred_element_type=jnp.float32)
        + b1_ref[...])
    o_ref[...] = jax.nn.sigmoid(
        jnp.dot(w2T_ref[...], h, preferred_element_type=jnp.float32)
        + b2_ref[...])


def _edge_mlp(tsumT, aT, btc, w1aT, w1bT, b1c, w2T, b2c, col0):
    blkc = 2560
    nblk = E // blkc
    aw = aT.shape[0]
    return pl.pallas_call(
        _mlp_body,
        grid=(nblk,),
        in_specs=[
            pl.BlockSpec((16, blkc), lambda i: (0, col0 + i)),
            pl.BlockSpec((aw, blkc), lambda i: (0, i)),
            pl.BlockSpec((16, 1), lambda i: (0, 0)),
            pl.BlockSpec((32, aw), lambda i: (0, 0)),
            pl.BlockSpec((32, 16), lambda i: (0, 0)),
            pl.BlockSpec((32, 1), lambda i: (0, 0)),
            pl.BlockSpec((1, 32), lambda i: (0, 0)),
            pl.BlockSpec((1, 1), lambda i: (0, 0)),
        ],
        out_specs=pl.BlockSpec((1, blkc), lambda i: (0, i)),
        out_shape=jax.ShapeDtypeStruct((1, E), jnp.float32),
    )(tsumT, aT, btc, w1aT, w1bT, b1c, w2T, b2c)


def _prop_body(src_hbm, dst_hbm, ew_hbm, m0_hbm, out_hbm,
               isrc, idst, wv, m_in, m_out, mrg, msl, sh_all, sh_merged):
    c = lax.axis_index("c")
    s = lax.axis_index("s")
    base = s * ET
    pltpu.sync_copy(src_hbm.at[c, pl.ds(base, ET)], isrc)
    pltpu.sync_copy(dst_hbm.at[c, pl.ds(base, ET)], idst)
    pltpu.sync_copy(ew_hbm.at[c, pl.ds(base, ET)], wv)
    pltpu.sync_copy(m0_hbm, m_in)

    def copy_m(i, carry):
        k = i * LANES
        m_out[pl.ds(k, LANES)] = m_in[pl.ds(k, LANES)]
        return carry

    def edge(i, carry):
        k = i * LANES
        si = isrc[pl.ds(k, LANES)]
        di = idst[pl.ds(k, LANES)]
        v = wv[pl.ds(k, LANES)] * plsc.load_gather(m_in, [si])

        def wbody(act):
            cur = plsc.load_gather(m_out, [di])
            plsc.store_scatter(m_out, [di], jnp.maximum(cur, v), mask=act)
            chk = plsc.load_gather(m_out, [di])
            return jnp.logical_and(act, chk < v)

        lax.while_loop(lambda a: jnp.any(a), wbody,
                       jnp.ones((LANES,), jnp.bool_))
        return carry

    def reduce_slice(i, carry):
        k = i * LANES
        acc = mrg[0, pl.ds(k, LANES)]
        for t in range(1, 16):
            acc = jnp.maximum(acc, mrg[t, pl.ds(k, LANES)])
        msl[pl.ds(k, LANES)] = acc
        return carry

    for rnd in range(3):
        lax.fori_loop(0, NP // LANES, copy_m, 0)
        lax.fori_loop(0, ET // LANES, edge, 0)
        pltpu.sync_copy(m_out, sh_all.at[s])
        plsc.subcore_barrier()
        for t in range(16):
            pltpu.sync_copy(sh_all.at[t, pl.ds(s * SL, SL)], mrg.at[t])
        lax.fori_loop(0, SL // LANES, reduce_slice, 0)
        if rnd < 2:
            pltpu.sync_copy(msl, sh_merged.at[pl.ds(s * SL, SL)])
            plsc.subcore_barrier()
            pltpu.sync_copy(sh_merged, m_in)
        else:
            pltpu.sync_copy(msl, out_hbm.at[c, pl.ds(s * SL, SL)])


def _propagate(src2, dst2, ew2, m0p):
    mesh = plsc.VectorSubcoreMesh(core_axis_name="c", subcore_axis_name="s")
    f = pl.kernel(
        _prop_body,
        out_type=jax.ShapeDtypeStruct((2, NP), jnp.float32),
        mesh=mesh,
        compiler_params=pltpu.CompilerParams(use_tc_tiling_on_sc=False, needs_layout_passes=False),
        scratch_types=[
            pltpu.VMEM((ET,), jnp.int32),
            pltpu.VMEM((ET,), jnp.int32),
            pltpu.VMEM((ET,), jnp.float32),
            pltpu.VMEM((NP,), jnp.float32),
            pltpu.VMEM((NP,), jnp.float32),
            pltpu.VMEM((16, SL), jnp.float32),
            pltpu.VMEM((SL,), jnp.float32),
            pltpu.VMEM_SHARED((16, NP), jnp.float32),
            pltpu.VMEM_SHARED((NP,), jnp.float32),
        ],
    )
    return f(src2, dst2, ew2, m0p)


def kernel(x, spatial_edge_index, spatial_edge_attr, dom_edge_index,
           dom_edge_attr, mask, Wt, bt, Wp1, bp1, Wp2, bp2, Wd1, bd1,
           Wd2, bd2):
    w = jnp.concatenate([Wt[:128], Wt[128:]], axis=1)
    xw = _node_matmul(x, w)
    xa = xw[:, :16]
    xb = xw[:, 16:]

    src = jnp.concatenate([spatial_edge_index[0], dom_edge_index[0]])
    dst = jnp.concatenate([spatial_edge_index[1], dom_edge_index[1]])

    tsum = _gather_sum(xa, xb, src, dst)

    tsumT = tsum.T
    btc = bt.reshape(16, 1)
    ew_s = _edge_mlp(tsumT, spatial_edge_attr.T, btc, Wp1[:4].T, Wp1[4:].T,
                     bp1.reshape(32, 1), Wp2.T, bp2.reshape(1, 1), 0)
    ew_d = _edge_mlp(tsumT, dom_edge_attr.T, btc, Wd1[:1].T, Wd1[1:].T,
                     bd1.reshape(32, 1), Wd2.T, bd2.reshape(1, 1), E // 2560)
    ew2 = jnp.concatenate([ew_s, ew_d], axis=0)

    src2 = src.reshape(2, E)
    dst2 = dst.reshape(2, E)
    m0p = jnp.pad(mask, (0, NP - N))
    mout = _propagate(src2, dst2, ew2, m0p)

    return jnp.maximum(mout[0, :N], mout[1, :N])


# speedup vs baseline: 2.0999x; 1.1869x over previous
import functools

import jax
import jax.numpy as jnp
from jax import lax
from jax.experimental import pallas as pl
from jax.experimental.pallas import tpu as pltpu
from jax.experimental.pallas import tpu_sc as plsc

N = 10000
E = 320000
NP = 10240
SL = NP // 16
ET = E // 16
EWK = 2 * E // 32
CH = 80
NCH = EWK // CH
LANES = 16


def _node_mm_body(x_ref, w_ref, o_ref):
    o_ref[...] = jnp.dot(x_ref[...], w_ref[...],
                         preferred_element_type=jnp.float32)


def _node_matmul(x, w):
    blk = 1000
    return pl.pallas_call(
        _node_mm_body,
        grid=(N // blk,),
        in_specs=[pl.BlockSpec((blk, 128), lambda i: (i, 0)),
                  pl.BlockSpec((128, 32), lambda i: (0, 0))],
        out_specs=pl.BlockSpec((blk, 32), lambda i: (i, 0)),
        out_shape=jax.ShapeDtypeStruct((N, 32), jnp.float32),
    )(x, w)


def _gather_sum_body(xa_hbm, xb_hbm, src_hbm, dst_hbm, out_hbm,
                     sidx, didx, abuf, bbuf, obuf, sema, semb, semo0, semo1):
    c = lax.axis_index("c")
    s = lax.axis_index("s")
    wid = c * 16 + s
    base = wid * EWK
    pltpu.sync_copy(src_hbm.at[pl.ds(base, EWK)], sidx)
    pltpu.sync_copy(dst_hbm.at[pl.ds(base, EWK)], didx)

    def fire(g, slot):
        pltpu.async_copy(xa_hbm.at[sidx.at[pl.ds(g * CH, CH)]],
                         abuf.at[slot], sema)
        pltpu.async_copy(xb_hbm.at[didx.at[pl.ds(g * CH, CH)]],
                         bbuf.at[slot], semb)

    fire(0, 0)
    fire(1, 1)

    def outer(i, carry):
        for b in range(2):
            g = i * 2 + b
            pltpu.make_async_copy(xa_hbm.at[sidx.at[pl.ds(0, CH)]],
                                  abuf.at[b], sema).wait()
            pltpu.make_async_copy(xb_hbm.at[didx.at[pl.ds(0, CH)]],
                                  bbuf.at[b], semb).wait()
            semo = semo0 if b == 0 else semo1

            @pl.when(g >= 2)
            def _():
                pltpu.make_async_copy(
                    obuf.at[b], out_hbm.at[pl.ds(base // 8, CH // 8)],
                    semo).wait()

            for r in range(CH):
                obuf[b, r // 8, pl.ds((r % 8) * 16, 16)] = (
                    abuf[b, r] + bbuf[b, r])
            pltpu.async_copy(
                obuf.at[b],
                out_hbm.at[pl.ds((base + g * CH) // 8, CH // 8)], semo)

            @pl.when(g + 2 < NCH)
            def _():
                fire(g + 2, b)
        return carry

    lax.fori_loop(0, NCH // 2, outer, 0)
    for b in range(2):
        semo = semo0 if b == 0 else semo1
        pltpu.make_async_copy(
            obuf.at[b], out_hbm.at[pl.ds(base // 8, CH // 8)], semo).wait()


def _gather_sum(xa, xb, src, dst):
    mesh = plsc.VectorSubcoreMesh(core_axis_name="c", subcore_axis_name="s")
    f = pl.kernel(
        _gather_sum_body,
        out_type=jax.ShapeDtypeStruct((2 * E // 8, 128), jnp.float32),
        mesh=mesh,
        compiler_params=pltpu.CompilerParams(use_tc_tiling_on_sc=False, needs_layout_passes=False),
        scratch_types=[
            pltpu.VMEM((EWK,), jnp.int32),
            pltpu.VMEM((EWK,), jnp.int32),
            pltpu.VMEM((2, CH, 16), jnp.float32),
            pltpu.VMEM((2, CH, 16), jnp.float32),
            pltpu.VMEM((2, CH // 8, 128), jnp.float32),
            pltpu.SemaphoreType.DMA,
            pltpu.SemaphoreType.DMA,
            pltpu.SemaphoreType.DMA,
            pltpu.SemaphoreType.DMA,
        ],
    )
    return f(xa, xb, src, dst)


def _mlp_body(t_ref, a_ref, btb_ref, w1ab_ref, w1bb_ref, b1b_ref, w2b_ref,
              b2b_ref, o_ref):
    trans = jax.nn.relu(t_ref[...] + btb_ref[...])
    h = jax.nn.relu(
        jnp.dot(a_ref[...], w1ab_ref[...], preferred_element_type=jnp.float32)
        + jnp.dot(trans, w1bb_ref[...], preferred_element_type=jnp.float32)
        + b1b_ref[...])
    o_ref[...] = jax.nn.sigmoid(
        jnp.dot(h, w2b_ref[...], preferred_element_type=jnp.float32)
        + b2b_ref[...])


def _edge_mlp(tpk, apk, btb, w1ab, w1bb, b1b, w2b, b2b, row0):
    brp = 320
    nblk = (E // 8) // brp
    aw = apk.shape[1]
    return pl.pallas_call(
        _mlp_body,
        grid=(nblk,),
        in_specs=[
            pl.BlockSpec((brp, 128), lambda i: (row0 + i, 0)),
            pl.BlockSpec((brp, aw), lambda i: (i, 0)),
            pl.BlockSpec((1, 128), lambda i: (0, 0)),
            pl.BlockSpec((aw, 256), lambda i: (0, 0)),
            pl.BlockSpec((128, 256), lambda i: (0, 0)),
            pl.BlockSpec((1, 256), lambda i: (0, 0)),
            pl.BlockSpec((256, 8), lambda i: (0, 0)),
            pl.BlockSpec((1, 8), lambda i: (0, 0)),
        ],
        out_specs=pl.BlockSpec((brp, 8), lambda i: (i, 0)),
        out_shape=jax.ShapeDtypeStruct((E // 8, 8), jnp.float32),
    )(tpk, apk, btb, w1ab, w1bb, b1b, w2b, b2b)


def _blockdiag(w):
    return jnp.kron(jnp.eye(8, dtype=w.dtype), w)

def _prop_body(src_hbm, dst_hbm, ew_hbm, m0_hbm, out_hbm,
               isrc, idst, wv, m_in, m_out, mrg, msl, sh_all, sh_merged):
    c = lax.axis_index("c")
    s = lax.axis_index("s")
    base = s * ET
    pltpu.sync_copy(src_hbm.at[c, pl.ds(base, ET)], isrc)
    pltpu.sync_copy(dst_hbm.at[c, pl.ds(base, ET)], idst)
    pltpu.sync_copy(ew_hbm.at[c, pl.ds(base, ET)], wv)
    pltpu.sync_copy(m0_hbm, m_in)

    def copy_m(i, carry):
        k = i * LANES
        m_out[pl.ds(k, LANES)] = m_in[pl.ds(k, LANES)]
        return carry

    def edge(i, carry):
        k = i * LANES
        si = isrc[pl.ds(k, LANES)]
        di = idst[pl.ds(k, LANES)]
        v = wv[pl.ds(k, LANES)] * plsc.load_gather(m_in, [si])

        def wbody(act):
            cur = plsc.load_gather(m_out, [di])
            plsc.store_scatter(m_out, [di], jnp.maximum(cur, v), mask=act)
            chk = plsc.load_gather(m_out, [di])
            return jnp.logical_and(act, chk < v)

        lax.while_loop(lambda a: jnp.any(a), wbody,
                       jnp.ones((LANES,), jnp.bool_))
        return carry

    def reduce_slice(i, carry):
        k = i * LANES
        acc = mrg[0, pl.ds(k, LANES)]
        for t in range(1, 16):
            acc = jnp.maximum(acc, mrg[t, pl.ds(k, LANES)])
        msl[pl.ds(k, LANES)] = acc
        return carry

    for rnd in range(3):
        lax.fori_loop(0, NP // LANES, copy_m, 0)
        lax.fori_loop(0, ET // LANES, edge, 0)
        pltpu.sync_copy(m_out, sh_all.at[s])
        plsc.subcore_barrier()
        for t in range(16):
            pltpu.sync_copy(sh_all.at[t, pl.ds(s * SL, SL)], mrg.at[t])
        lax.fori_loop(0, SL // LANES, reduce_slice, 0)
        if rnd < 2:
            pltpu.sync_copy(msl, sh_merged.at[pl.ds(s * SL, SL)])
            plsc.subcore_barrier()
            pltpu.sync_copy(sh_merged, m_in)
        else:
            pltpu.sync_copy(msl, out_hbm.at[c, pl.ds(s * SL, SL)])


def _propagate(src2, dst2, ew2, m0p):
    mesh = plsc.VectorSubcoreMesh(core_axis_name="c", subcore_axis_name="s")
    f = pl.kernel(
        _prop_body,
        out_type=jax.ShapeDtypeStruct((2, NP), jnp.float32),
        mesh=mesh,
        compiler_params=pltpu.CompilerParams(use_tc_tiling_on_sc=False, needs_layout_passes=False),
        scratch_types=[
            pltpu.VMEM((ET,), jnp.int32),
            pltpu.VMEM((ET,), jnp.int32),
            pltpu.VMEM((ET,), jnp.float32),
            pltpu.VMEM((NP,), jnp.float32),
            pltpu.VMEM((NP,), jnp.float32),
            pltpu.VMEM((16, SL), jnp.float32),
            pltpu.VMEM((SL,), jnp.float32),
            pltpu.VMEM_SHARED((16, NP), jnp.float32),
            pltpu.VMEM_SHARED((NP,), jnp.float32),
        ],
    )
    return f(src2, dst2, ew2, m0p)


def kernel(x, spatial_edge_index, spatial_edge_attr, dom_edge_index,
           dom_edge_attr, mask, Wt, bt, Wp1, bp1, Wp2, bp2, Wd1, bd1,
           Wd2, bd2):
    w = jnp.concatenate([Wt[:128], Wt[128:]], axis=1)
    xw = _node_matmul(x, w)
    xa = xw[:, :16]
    xb = xw[:, 16:]

    src = jnp.concatenate([spatial_edge_index[0], dom_edge_index[0]])
    dst = jnp.concatenate([spatial_edge_index[1], dom_edge_index[1]])

    tsum = _gather_sum(xa, xb, src, dst)

    btb = jnp.tile(bt, 8).reshape(1, 128)
    b1b = jnp.tile(bp1, 8).reshape(1, 256)
    b1d = jnp.tile(bd1, 8).reshape(1, 256)
    ew_s = _edge_mlp(tsum, spatial_edge_attr.reshape(E // 8, 32), btb,
                     _blockdiag(Wp1[:4]), _blockdiag(Wp1[4:]), b1b,
                     _blockdiag(Wp2), jnp.full((1, 8), bp2[0]), 0)
    ew_d = _edge_mlp(tsum, dom_edge_attr.reshape(E // 8, 8), btb,
                     _blockdiag(Wd1[:1]), _blockdiag(Wd1[1:]), b1d,
                     _blockdiag(Wd2), jnp.full((1, 8), bd2[0]), (E // 8) // 320)
    ew2 = jnp.stack([ew_s.reshape(E), ew_d.reshape(E)])

    src2 = src.reshape(2, E)
    dst2 = dst.reshape(2, E)
    m0p = jnp.pad(mask, (0, NP - N))
    mout = _propagate(src2, dst2, ew2, m0p)

    return jnp.maximum(mout[0, :N], mout[1, :N])
